# baseline (speedup 1.0000x reference)
import functools

import jax
import jax.numpy as jnp
from jax.experimental import pallas as pl
from jax.experimental.pallas import tpu as pltpu

_EPS = 1e-5
_F32 = jnp.float32
_BF16 = jnp.bfloat16


def _relu(v):
    return jnp.maximum(v, 0.0)


def _fold(st, g_ref, b_ref, inv_count):
    s = jnp.sum(st[:, 0:1, :], axis=0)
    ss = jnp.sum(st[:, 1:2, :], axis=0)
    mean = s * inv_count
    var = jnp.maximum(ss * inv_count - mean * mean, 0.0)
    scale = g_ref[...] * jax.lax.rsqrt(var + _EPS)
    offset = b_ref[...] - mean * scale
    return scale, offset


def _stage1_conv1(x_ref, w1_ref, y1_ref, st_ref):
    tn, M, Cin = x_ref.shape
    xf = x_ref[...].reshape(tn * M, Cin)
    y = jnp.dot(xf, w1_ref[...], preferred_element_type=_F32)
    y1_ref[...] = y.reshape(tn, M, -1).astype(_BF16)
    s = jnp.sum(y, axis=0, keepdims=True)
    ss = jnp.sum(y * y, axis=0, keepdims=True)
    st_ref[...] = jnp.concatenate([s, ss], axis=0)[None]


def _stage2_conv2(y1_ref, st1_ref, g1_ref, b1_ref, w2_ref, y2_ref, st_ref,
                  pad_ref, *, inv_count, H, W):
    tn, M, C = y1_ref.shape
    sc1, of1 = _fold(st1_ref[...], g1_ref, b1_ref, inv_count)
    a = y1_ref[...].astype(_F32).reshape(tn * M, C)
    a = _relu(a * sc1 + of1)

    pad_ref[:, 0:1, :, :] = jnp.zeros((tn, 1, W + 2, C), _F32)
    pad_ref[:, H + 1:H + 2, :, :] = jnp.zeros((tn, 1, W + 2, C), _F32)
    pad_ref[:, :, 0:1, :] = jnp.zeros((tn, H + 2, 1, C), _F32)
    pad_ref[:, :, W + 1:W + 2, :] = jnp.zeros((tn, H + 2, 1, C), _F32)
    pad_ref[:, 1:H + 1, 1:W + 1, :] = a.reshape(tn, H, W, C)

    rows = [pad_ref[:, dh:dh + H, :, :].reshape(tn * H * (W + 2), C)
            for dh in range(3)]
    p = jnp.concatenate(rows, axis=1)
    u = jnp.dot(p, w2_ref[...], preferred_element_type=_F32)
    u = u.reshape(tn, H, W + 2, 3 * C)
    y = (u[:, :, 0:W, 0:C] + u[:, :, 1:W + 1, C:2 * C]
         + u[:, :, 2:W + 2, 2 * C:3 * C]).reshape(tn * M, C)
    y2_ref[...] = y.reshape(tn, M, C).astype(_BF16)
    s = jnp.sum(y, axis=0, keepdims=True)
    ss = jnp.sum(y * y, axis=0, keepdims=True)
    st_ref[...] = jnp.concatenate([s, ss], axis=0)[None]


def _stage3_stats(y2_ref, st2_ref, g2_ref, b2_ref, s_ref, g_ref, *,
                  inv_count):
    tn, M, C = y2_ref.shape
    sc2, of2 = _fold(st2_ref[...], g2_ref, b2_ref, inv_count)
    a = y2_ref[...].astype(_F32).reshape(tn * M, C)
    a = _relu(a * sc2 + of2)
    s_ref[...] = jnp.sum(a, axis=0, keepdims=True)[None]
    ab = a.astype(_BF16)
    g_ref[...] = jax.lax.dot_general(
        ab, ab, (((0,), (0,)), ((), ())),
        preferred_element_type=_F32)[None]


def _stage4_out(y2_ref, st2_ref, g2_ref, b2_ref, w3_ref, s3_ref, gr3_ref,
                g3_ref, b3_ref, x_ref, o_ref, *, inv_count):
    tn, M, C = y2_ref.shape
    sc2, of2 = _fold(st2_ref[...], g2_ref, b2_ref, inv_count)

    w3 = w3_ref[...]
    asum = jnp.sum(s3_ref[:, 0, :], axis=0, keepdims=True)
    gram = jnp.sum(gr3_ref[...], axis=0)
    mean3 = jnp.dot(asum, w3, preferred_element_type=_F32) * inv_count
    ey3sq = jnp.sum(jnp.dot(gram, w3, preferred_element_type=_F32) * w3,
                    axis=0, keepdims=True) * inv_count
    var3 = jnp.maximum(ey3sq - mean3 * mean3, 0.0)
    sc3 = g3_ref[...] * jax.lax.rsqrt(var3 + _EPS)
    of3 = b3_ref[...] - mean3 * sc3

    a = y2_ref[...].astype(_F32).reshape(tn * M, C)
    a = _relu(a * sc2 + of2)
    y = jnp.dot(a, w3, preferred_element_type=_F32)
    y = y * sc3 + of3 + x_ref[...].reshape(tn * M, -1)
    o_ref[...] = _relu(y).reshape(tn, M, -1).astype(o_ref.dtype)


def kernel(x, w1, g1, b1, w2, g2, b2, w3, g3, b3):
    N, H, W, Cin = x.shape
    Cmid = w1.shape[1]
    Cout = w3.shape[1]
    M = H * W
    inv_count = 1.0 / float(N * M)

    xf = x.reshape(N, M, Cin)
    w1f = w1.astype(_F32)
    w2f = jnp.transpose(w2.astype(_F32), (0, 2, 1, 3)).reshape(
        3 * Cmid, 3 * Cmid)
    w3f = w3.astype(_F32)
    g1f, b1f = g1.reshape(1, Cmid), b1.reshape(1, Cmid)
    g2f, b2f = g2.reshape(1, Cmid), b2.reshape(1, Cmid)
    g3f, b3f = g3.reshape(1, Cout), b3.reshape(1, Cout)

    tile_n = 1
    ntiles = N // tile_n
    params = pltpu.CompilerParams(
        dimension_semantics=("parallel",),
        vmem_limit_bytes=48 * 1024 * 1024,
    )
    vec_spec = pl.BlockSpec((1, Cmid), lambda i: (0, 0))
    act_spec = pl.BlockSpec((tile_n, M, Cmid), lambda i: (i, 0, 0))
    st_spec = pl.BlockSpec((ntiles, 2, Cmid), lambda i: (0, 0, 0))

    y1, st1 = pl.pallas_call(
        _stage1_conv1,
        grid=(ntiles,),
        in_specs=[
            pl.BlockSpec((tile_n, M, Cin), lambda i: (i, 0, 0)),
            pl.BlockSpec((Cin, Cmid), lambda i: (0, 0)),
        ],
        out_specs=(
            act_spec,
            pl.BlockSpec((1, 2, Cmid), lambda i: (i, 0, 0)),
        ),
        out_shape=(
            jax.ShapeDtypeStruct((N, M, Cmid), _BF16),
            jax.ShapeDtypeStruct((ntiles, 2, Cmid), _F32),
        ),
        compiler_params=params,
        cost_estimate=pl.CostEstimate(
            flops=2 * N * M * Cin * Cmid, transcendentals=0,
            bytes_accessed=4 * N * M * Cin + 2 * N * M * Cmid),
    )(xf, w1f)

    y2, st2 = pl.pallas_call(
        functools.partial(_stage2_conv2, inv_count=inv_count, H=H, W=W),
        grid=(ntiles,),
        in_specs=[
            act_spec,
            st_spec,
            pl.BlockSpec((1, Cmid), lambda i: (0, 0)),
            pl.BlockSpec((1, Cmid), lambda i: (0, 0)),
            pl.BlockSpec((3 * Cmid, 3 * Cmid), lambda i: (0, 0)),
        ],
        out_specs=(
            act_spec,
            pl.BlockSpec((1, 2, Cmid), lambda i: (i, 0, 0)),
        ),
        out_shape=(
            jax.ShapeDtypeStruct((N, M, Cmid), _BF16),
            jax.ShapeDtypeStruct((ntiles, 2, Cmid), _F32),
        ),
        scratch_shapes=[pltpu.VMEM((tile_n, H + 2, W + 2, Cmid), _F32)],
        compiler_params=params,
        cost_estimate=pl.CostEstimate(
            flops=2 * N * M * 9 * Cmid * Cmid, transcendentals=0,
            bytes_accessed=4 * N * M * Cmid),
    )(y1, st1, g1f, b1f, w2f)

    s3, gr3 = pl.pallas_call(
        functools.partial(_stage3_stats, inv_count=inv_count),
        grid=(ntiles,),
        in_specs=[
            act_spec,
            st_spec, vec_spec, vec_spec,
        ],
        out_specs=(
            pl.BlockSpec((1, 1, Cmid), lambda i: (i, 0, 0)),
            pl.BlockSpec((1, Cmid, Cmid), lambda i: (i, 0, 0)),
        ),
        out_shape=(
            jax.ShapeDtypeStruct((ntiles, 1, Cmid), _F32),
            jax.ShapeDtypeStruct((ntiles, Cmid, Cmid), _F32),
        ),
        compiler_params=params,
        cost_estimate=pl.CostEstimate(
            flops=2 * N * M * Cmid * Cmid, transcendentals=0,
            bytes_accessed=2 * N * M * Cmid),
    )(y2, st2, g2f, b2f)

    out = pl.pallas_call(
        functools.partial(_stage4_out, inv_count=inv_count),
        grid=(ntiles,),
        in_specs=[
            act_spec,
            st_spec, vec_spec, vec_spec,
            pl.BlockSpec((Cmid, Cout), lambda i: (0, 0)),
            pl.BlockSpec((ntiles, 1, Cmid), lambda i: (0, 0, 0)),
            pl.BlockSpec((ntiles, Cmid, Cmid), lambda i: (0, 0, 0)),
            pl.BlockSpec((1, Cout), lambda i: (0, 0)),
            pl.BlockSpec((1, Cout), lambda i: (0, 0)),
            pl.BlockSpec((tile_n, M, Cin), lambda i: (i, 0, 0)),
        ],
        out_specs=pl.BlockSpec((tile_n, M, Cout), lambda i: (i, 0, 0)),
        out_shape=jax.ShapeDtypeStruct((N, M, Cout), x.dtype),
        compiler_params=params,
        cost_estimate=pl.CostEstimate(
            flops=2 * N * M * Cmid * Cout + 4 * N * M * Cout,
            transcendentals=0,
            bytes_accessed=2 * N * M * Cmid + 8 * N * M * Cout),
    )(y2, st2, g2f, b2f, w3f, s3, gr3, g3f, b3f, xf)

    return out.reshape(N, H, W, Cout)


# speedup vs baseline: 1.5163x; 1.5163x over previous
import functools

import jax
import jax.numpy as jnp
from jax.experimental import pallas as pl
from jax.experimental.pallas import tpu as pltpu

_EPS = 1e-5
_F32 = jnp.float32
_BF16 = jnp.bfloat16


def _relu(v):
    return jnp.maximum(v, 0.0)


def _fold(st, g_ref, b_ref, inv_count):
    s = jnp.sum(st[:, 0:1, :], axis=0)
    ss = jnp.sum(st[:, 1:2, :], axis=0)
    mean = s * inv_count
    var = jnp.maximum(ss * inv_count - mean * mean, 0.0)
    scale = g_ref[...] * jax.lax.rsqrt(var + _EPS)
    offset = b_ref[...] - mean * scale
    return scale, offset


def _stage1_conv1(x_ref, w1_ref, y1_ref, st_ref):
    tn, M, Cin = x_ref.shape
    xf = x_ref[...].reshape(tn * M, Cin)
    y = jnp.dot(xf, w1_ref[...], preferred_element_type=_F32)
    y1_ref[...] = y.reshape(tn, M, -1).astype(_BF16)
    s = jnp.sum(y, axis=0, keepdims=True)
    ss = jnp.sum(y * y, axis=0, keepdims=True)
    st_ref[...] = jnp.concatenate([s, ss], axis=0)[None]


def _stage2_conv2(y1_ref, st1_ref, g1_ref, b1_ref, w2_ref, y2_ref, st_ref,
                  pad_ref, *, inv_count, H, W):
    tn, M, C = y1_ref.shape
    sc1, of1 = _fold(st1_ref[...], g1_ref, b1_ref, inv_count)
    a = y1_ref[...].astype(_F32).reshape(tn * M, C)
    a = _relu(a * sc1 + of1)

    pad_ref[:, 0:1, :, :] = jnp.zeros((tn, 1, W + 2, C), _F32)
    pad_ref[:, H + 1:H + 2, :, :] = jnp.zeros((tn, 1, W + 2, C), _F32)
    pad_ref[:, :, 0:1, :] = jnp.zeros((tn, H + 2, 1, C), _F32)
    pad_ref[:, :, W + 1:W + 2, :] = jnp.zeros((tn, H + 2, 1, C), _F32)
    pad_ref[:, 1:H + 1, 1:W + 1, :] = a.reshape(tn, H, W, C)

    taps = [pad_ref[:, dh:dh + H, dw:dw + W, :].reshape(tn * M, C)
            for dh in range(3) for dw in range(3)]
    patches = jnp.concatenate(taps, axis=1)
    y = jnp.dot(patches, w2_ref[...], preferred_element_type=_F32)
    y2_ref[...] = y.reshape(tn, M, C).astype(_BF16)
    s = jnp.sum(y, axis=0, keepdims=True)
    ss = jnp.sum(y * y, axis=0, keepdims=True)
    st_ref[...] = jnp.concatenate([s, ss], axis=0)[None]


def _stage3_stats(y2_ref, st2_ref, g2_ref, b2_ref, s_ref, g_ref, *,
                  inv_count):
    tn, M, C = y2_ref.shape
    sc2, of2 = _fold(st2_ref[...], g2_ref, b2_ref, inv_count)
    a = y2_ref[...].astype(_F32).reshape(tn * M, C)
    a = _relu(a * sc2 + of2)
    s_ref[...] = jnp.sum(a, axis=0, keepdims=True)[None]
    g_ref[...] = jax.lax.dot_general(
        a, a, (((0,), (0,)), ((), ())),
        preferred_element_type=_F32)[None]


def _stage4_out(y2_ref, st2_ref, g2_ref, b2_ref, w3_ref, s3_ref, gr3_ref,
                g3_ref, b3_ref, x_ref, o_ref, *, inv_count):
    tn, M, C = y2_ref.shape
    sc2, of2 = _fold(st2_ref[...], g2_ref, b2_ref, inv_count)

    w3 = w3_ref[...]
    asum = jnp.sum(s3_ref[:, 0, :], axis=0, keepdims=True)
    gram = jnp.sum(gr3_ref[...], axis=0)
    mean3 = jnp.dot(asum, w3, preferred_element_type=_F32) * inv_count
    ey3sq = jnp.sum(jnp.dot(gram, w3, preferred_element_type=_F32) * w3,
                    axis=0, keepdims=True) * inv_count
    var3 = jnp.maximum(ey3sq - mean3 * mean3, 0.0)
    sc3 = g3_ref[...] * jax.lax.rsqrt(var3 + _EPS)
    of3 = b3_ref[...] - mean3 * sc3

    a = y2_ref[...].astype(_F32).reshape(tn * M, C)
    a = _relu(a * sc2 + of2)
    y = jnp.dot(a, w3, preferred_element_type=_F32)
    y = y * sc3 + of3 + x_ref[...].reshape(tn * M, -1)
    o_ref[...] = _relu(y).reshape(tn, M, -1).astype(o_ref.dtype)


def kernel(x, w1, g1, b1, w2, g2, b2, w3, g3, b3):
    N, H, W, Cin = x.shape
    Cmid = w1.shape[1]
    Cout = w3.shape[1]
    M = H * W
    inv_count = 1.0 / float(N * M)

    xf = x.reshape(N, M, Cin)
    w1f = w1.astype(_F32)
    w2f = w2.astype(_F32).reshape(9 * Cmid, Cmid)
    w3f = w3.astype(_F32)
    g1f, b1f = g1.reshape(1, Cmid), b1.reshape(1, Cmid)
    g2f, b2f = g2.reshape(1, Cmid), b2.reshape(1, Cmid)
    g3f, b3f = g3.reshape(1, Cout), b3.reshape(1, Cout)

    tile_n = 1
    ntiles = N // tile_n
    params = pltpu.CompilerParams(
        dimension_semantics=("parallel",),
        vmem_limit_bytes=48 * 1024 * 1024,
    )
    vec_spec = pl.BlockSpec((1, Cmid), lambda i: (0, 0))
    act_spec = pl.BlockSpec((tile_n, M, Cmid), lambda i: (i, 0, 0))
    st_spec = pl.BlockSpec((ntiles, 2, Cmid), lambda i: (0, 0, 0))

    y1, st1 = pl.pallas_call(
        _stage1_conv1,
        grid=(ntiles,),
        in_specs=[
            pl.BlockSpec((tile_n, M, Cin), lambda i: (i, 0, 0)),
            pl.BlockSpec((Cin, Cmid), lambda i: (0, 0)),
        ],
        out_specs=(
            act_spec,
            pl.BlockSpec((1, 2, Cmid), lambda i: (i, 0, 0)),
        ),
        out_shape=(
            jax.ShapeDtypeStruct((N, M, Cmid), _BF16),
            jax.ShapeDtypeStruct((ntiles, 2, Cmid), _F32),
        ),
        compiler_params=params,
        cost_estimate=pl.CostEstimate(
            flops=2 * N * M * Cin * Cmid, transcendentals=0,
            bytes_accessed=4 * N * M * Cin + 2 * N * M * Cmid),
    )(xf, w1f)

    y2, st2 = pl.pallas_call(
        functools.partial(_stage2_conv2, inv_count=inv_count, H=H, W=W),
        grid=(ntiles,),
        in_specs=[
            act_spec,
            st_spec,
            pl.BlockSpec((1, Cmid), lambda i: (0, 0)),
            pl.BlockSpec((1, Cmid), lambda i: (0, 0)),
            pl.BlockSpec((9 * Cmid, Cmid), lambda i: (0, 0)),
        ],
        out_specs=(
            act_spec,
            pl.BlockSpec((1, 2, Cmid), lambda i: (i, 0, 0)),
        ),
        out_shape=(
            jax.ShapeDtypeStruct((N, M, Cmid), _BF16),
            jax.ShapeDtypeStruct((ntiles, 2, Cmid), _F32),
        ),
        scratch_shapes=[pltpu.VMEM((tile_n, H + 2, W + 2, Cmid), _F32)],
        compiler_params=params,
        cost_estimate=pl.CostEstimate(
            flops=2 * N * M * 9 * Cmid * Cmid, transcendentals=0,
            bytes_accessed=4 * N * M * Cmid),
    )(y1, st1, g1f, b1f, w2f)

    s3, gr3 = pl.pallas_call(
        functools.partial(_stage3_stats, inv_count=inv_count),
        grid=(ntiles,),
        in_specs=[
            act_spec,
            st_spec, vec_spec, vec_spec,
        ],
        out_specs=(
            pl.BlockSpec((1, 1, Cmid), lambda i: (i, 0, 0)),
            pl.BlockSpec((1, Cmid, Cmid), lambda i: (i, 0, 0)),
        ),
        out_shape=(
            jax.ShapeDtypeStruct((ntiles, 1, Cmid), _F32),
            jax.ShapeDtypeStruct((ntiles, Cmid, Cmid), _F32),
        ),
        compiler_params=params,
        cost_estimate=pl.CostEstimate(
            flops=2 * N * M * Cmid * Cmid, transcendentals=0,
            bytes_accessed=2 * N * M * Cmid),
    )(y2, st2, g2f, b2f)

    out = pl.pallas_call(
        functools.partial(_stage4_out, inv_count=inv_count),
        grid=(ntiles,),
        in_specs=[
            act_spec,
            st_spec, vec_spec, vec_spec,
            pl.BlockSpec((Cmid, Cout), lambda i: (0, 0)),
            pl.BlockSpec((ntiles, 1, Cmid), lambda i: (0, 0, 0)),
            pl.BlockSpec((ntiles, Cmid, Cmid), lambda i: (0, 0, 0)),
            pl.BlockSpec((1, Cout), lambda i: (0, 0)),
            pl.BlockSpec((1, Cout), lambda i: (0, 0)),
            pl.BlockSpec((tile_n, M, Cin), lambda i: (i, 0, 0)),
        ],
        out_specs=pl.BlockSpec((tile_n, M, Cout), lambda i: (i, 0, 0)),
        out_shape=jax.ShapeDtypeStruct((N, M, Cout), x.dtype),
        compiler_params=params,
        cost_estimate=pl.CostEstimate(
            flops=2 * N * M * Cmid * Cout + 4 * N * M * Cout,
            transcendentals=0,
            bytes_accessed=2 * N * M * Cmid + 8 * N * M * Cout),
    )(y2, st2, g2f, b2f, w3f, s3, gr3, g3f, b3f, xf)

    return out.reshape(N, H, W, Cout)


# speedup vs baseline: 1.6708x; 1.1019x over previous
import functools

import jax
import jax.numpy as jnp
from jax.experimental import pallas as pl
from jax.experimental.pallas import tpu as pltpu

_EPS = 1e-5
_F32 = jnp.float32
_BF16 = jnp.bfloat16


def _relu(v):
    return jnp.maximum(v, 0.0)


def _fold(st, g_ref, b_ref, inv_count):
    s = jnp.sum(st[:, 0:1, :], axis=0)
    ss = jnp.sum(st[:, 1:2, :], axis=0)
    mean = s * inv_count
    var = jnp.maximum(ss * inv_count - mean * mean, 0.0)
    scale = g_ref[...] * jax.lax.rsqrt(var + _EPS)
    offset = b_ref[...] - mean * scale
    return scale, offset


def _stage1_conv1(x_ref, w1_ref, y1_ref, st_ref):
    tn, M, Cin = x_ref.shape
    xf = x_ref[...].reshape(tn * M, Cin)
    y = jnp.dot(xf, w1_ref[...], preferred_element_type=_F32)
    y1_ref[...] = y.reshape(tn, M, -1).astype(_BF16)
    s = jnp.sum(y, axis=0, keepdims=True)
    ss = jnp.sum(y * y, axis=0, keepdims=True)
    st_ref[...] = jnp.concatenate([s, ss], axis=0)[None]


def _stage2_conv2(y1_ref, st1_ref, g1_ref, b1_ref, w2_ref, y2_ref, st_ref,
                  pad_ref, *, inv_count, H, W):
    tn, M, C = y1_ref.shape
    sc1, of1 = _fold(st1_ref[...], g1_ref, b1_ref, inv_count)
    a = y1_ref[...].astype(_F32).reshape(tn * M, C)
    a = _relu(a * sc1 + of1)

    pad_ref[:, 0:1, :, :] = jnp.zeros((tn, 1, W + 2, C), _F32)
    pad_ref[:, H + 1:H + 2, :, :] = jnp.zeros((tn, 1, W + 2, C), _F32)
    pad_ref[:, :, 0:1, :] = jnp.zeros((tn, H + 2, 1, C), _F32)
    pad_ref[:, :, W + 1:W + 2, :] = jnp.zeros((tn, H + 2, 1, C), _F32)
    pad_ref[:, 1:H + 1, 1:W + 1, :] = a.reshape(tn, H, W, C)

    taps = [pad_ref[:, dh:dh + H, dw:dw + W, :].reshape(tn * M, C)
            for dh in range(3) for dw in range(3)]
    patches = jnp.concatenate(taps, axis=1)
    y = jnp.dot(patches, w2_ref[...], preferred_element_type=_F32)
    y2_ref[...] = y.reshape(tn, M, C).astype(_BF16)
    s = jnp.sum(y, axis=0, keepdims=True)
    ss = jnp.sum(y * y, axis=0, keepdims=True)
    st_ref[...] = jnp.concatenate([s, ss], axis=0)[None]


def _stage3_stats(y2_ref, st2_ref, g2_ref, b2_ref, s_ref, g_ref, *,
                  inv_count):
    tn, M, C = y2_ref.shape
    sc2, of2 = _fold(st2_ref[...], g2_ref, b2_ref, inv_count)
    a = y2_ref[...].astype(_F32).reshape(tn * M, C)
    a = _relu(a * sc2 + of2)
    s_ref[...] = jnp.sum(a, axis=0, keepdims=True)[None]
    g_ref[...] = jax.lax.dot_general(
        a, a, (((0,), (0,)), ((), ())),
        preferred_element_type=_F32)[None]


def _stage4_out(y2_ref, st2_ref, g2_ref, b2_ref, w3_ref, s3_ref, gr3_ref,
                g3_ref, b3_ref, x_ref, o_ref, *, inv_count):
    tn, M, C = y2_ref.shape
    sc2, of2 = _fold(st2_ref[...], g2_ref, b2_ref, inv_count)

    w3 = w3_ref[...]
    asum = jnp.sum(s3_ref[:, 0, :], axis=0, keepdims=True)
    gram = jnp.sum(gr3_ref[...], axis=0)
    mean3 = jnp.dot(asum, w3, preferred_element_type=_F32) * inv_count
    ey3sq = jnp.sum(jnp.dot(gram, w3, preferred_element_type=_F32) * w3,
                    axis=0, keepdims=True) * inv_count
    var3 = jnp.maximum(ey3sq - mean3 * mean3, 0.0)
    sc3 = g3_ref[...] * jax.lax.rsqrt(var3 + _EPS)
    of3 = b3_ref[...] - mean3 * sc3

    a = y2_ref[...].astype(_F32).reshape(tn * M, C)
    a = _relu(a * sc2 + of2)
    y = jnp.dot(a, w3, preferred_element_type=_F32)
    y = y * sc3 + of3 + x_ref[...].reshape(tn * M, -1)
    o_ref[...] = _relu(y).reshape(tn, M, -1).astype(o_ref.dtype)


def kernel(x, w1, g1, b1, w2, g2, b2, w3, g3, b3):
    N, H, W, Cin = x.shape
    Cmid = w1.shape[1]
    Cout = w3.shape[1]
    M = H * W
    inv_count = 1.0 / float(N * M)

    xf = x.reshape(N, M, Cin)
    w1f = w1.astype(_F32)
    w2f = w2.astype(_F32).reshape(9 * Cmid, Cmid)
    w3f = w3.astype(_F32)
    g1f, b1f = g1.reshape(1, Cmid), b1.reshape(1, Cmid)
    g2f, b2f = g2.reshape(1, Cmid), b2.reshape(1, Cmid)
    g3f, b3f = g3.reshape(1, Cout), b3.reshape(1, Cout)

    tile_n = 2
    ntiles = N // tile_n
    params = pltpu.CompilerParams(
        dimension_semantics=("parallel",),
        vmem_limit_bytes=48 * 1024 * 1024,
    )
    vec_spec = pl.BlockSpec((1, Cmid), lambda i: (0, 0))
    act_spec = pl.BlockSpec((tile_n, M, Cmid), lambda i: (i, 0, 0))
    st_spec = pl.BlockSpec((ntiles, 2, Cmid), lambda i: (0, 0, 0))

    y1, st1 = pl.pallas_call(
        _stage1_conv1,
        grid=(ntiles,),
        in_specs=[
            pl.BlockSpec((tile_n, M, Cin), lambda i: (i, 0, 0)),
            pl.BlockSpec((Cin, Cmid), lambda i: (0, 0)),
        ],
        out_specs=(
            act_spec,
            pl.BlockSpec((1, 2, Cmid), lambda i: (i, 0, 0)),
        ),
        out_shape=(
            jax.ShapeDtypeStruct((N, M, Cmid), _BF16),
            jax.ShapeDtypeStruct((ntiles, 2, Cmid), _F32),
        ),
        compiler_params=params,
        cost_estimate=pl.CostEstimate(
            flops=2 * N * M * Cin * Cmid, transcendentals=0,
            bytes_accessed=4 * N * M * Cin + 2 * N * M * Cmid),
    )(xf, w1f)

    y2, st2 = pl.pallas_call(
        functools.partial(_stage2_conv2, inv_count=inv_count, H=H, W=W),
        grid=(ntiles,),
        in_specs=[
            act_spec,
            st_spec,
            pl.BlockSpec((1, Cmid), lambda i: (0, 0)),
            pl.BlockSpec((1, Cmid), lambda i: (0, 0)),
            pl.BlockSpec((9 * Cmid, Cmid), lambda i: (0, 0)),
        ],
        out_specs=(
            act_spec,
            pl.BlockSpec((1, 2, Cmid), lambda i: (i, 0, 0)),
        ),
        out_shape=(
            jax.ShapeDtypeStruct((N, M, Cmid), _BF16),
            jax.ShapeDtypeStruct((ntiles, 2, Cmid), _F32),
        ),
        scratch_shapes=[pltpu.VMEM((tile_n, H + 2, W + 2, Cmid), _F32)],
        compiler_params=params,
        cost_estimate=pl.CostEstimate(
            flops=2 * N * M * 9 * Cmid * Cmid, transcendentals=0,
            bytes_accessed=4 * N * M * Cmid),
    )(y1, st1, g1f, b1f, w2f)

    s3, gr3 = pl.pallas_call(
        functools.partial(_stage3_stats, inv_count=inv_count),
        grid=(ntiles,),
        in_specs=[
            act_spec,
            st_spec, vec_spec, vec_spec,
        ],
        out_specs=(
            pl.BlockSpec((1, 1, Cmid), lambda i: (i, 0, 0)),
            pl.BlockSpec((1, Cmid, Cmid), lambda i: (i, 0, 0)),
        ),
        out_shape=(
            jax.ShapeDtypeStruct((ntiles, 1, Cmid), _F32),
            jax.ShapeDtypeStruct((ntiles, Cmid, Cmid), _F32),
        ),
        compiler_params=params,
        cost_estimate=pl.CostEstimate(
            flops=2 * N * M * Cmid * Cmid, transcendentals=0,
            bytes_accessed=2 * N * M * Cmid),
    )(y2, st2, g2f, b2f)

    out = pl.pallas_call(
        functools.partial(_stage4_out, inv_count=inv_count),
        grid=(ntiles,),
        in_specs=[
            act_spec,
            st_spec, vec_spec, vec_spec,
            pl.BlockSpec((Cmid, Cout), lambda i: (0, 0)),
            pl.BlockSpec((ntiles, 1, Cmid), lambda i: (0, 0, 0)),
            pl.BlockSpec((ntiles, Cmid, Cmid), lambda i: (0, 0, 0)),
            pl.BlockSpec((1, Cout), lambda i: (0, 0)),
            pl.BlockSpec((1, Cout), lambda i: (0, 0)),
            pl.BlockSpec((tile_n, M, Cin), lambda i: (i, 0, 0)),
        ],
        out_specs=pl.BlockSpec((tile_n, M, Cout), lambda i: (i, 0, 0)),
        out_shape=jax.ShapeDtypeStruct((N, M, Cout), x.dtype),
        compiler_params=params,
        cost_estimate=pl.CostEstimate(
            flops=2 * N * M * Cmid * Cout + 4 * N * M * Cout,
            transcendentals=0,
            bytes_accessed=2 * N * M * Cmid + 8 * N * M * Cout),
    )(y2, st2, g2f, b2f, w3f, s3, gr3, g3f, b3f, xf)

    return out.reshape(N, H, W, Cout)
